# Initial kernel scaffold; baseline (speedup 1.0000x reference)
import jax
import jax.numpy as jnp
from jax.experimental import pallas as pl


def kernel(x, edge_index, edge_weight, W1, b1, W2, b2):
    raise NotImplementedError("write your pallas kernel here")


# speedup vs baseline: 14.3691x; 14.3691x over previous
import functools
import jax
import jax.numpy as jnp
from jax import lax
from jax.experimental import pallas as pl
from jax.experimental.pallas import tpu as pltpu
from jax.experimental.pallas import tpu_sc as plsc

N = 10000
E = 320000
D = 128

NC = 2
NS = 16
NW = NC * NS

NPAD = 10240
EPAD = 327680
ROWS_PER_TILE = NPAD // NS
ECH = EPAD // (NW * 128)
DEG_CH = EPAD // (NS * 128)

_mesh = plsc.VectorSubcoreMesh(
    core_axis_name="c", subcore_axis_name="s", num_cores=NC, num_subcores=NS)

_sc_params = pltpu.CompilerParams(needs_layout_passes=False)


def _z16():
    return jnp.zeros((16,), jnp.float32)


@functools.partial(
    pl.kernel,
    out_type=jax.ShapeDtypeStruct((NPAD,), jnp.float32),
    mesh=_mesh,
    scratch_types=[
        pltpu.VMEM_SHARED((NPAD,), jnp.float32),
        pltpu.VMEM((DEG_CH, 128), jnp.int32),
        pltpu.VMEM((DEG_CH, 128), jnp.float32),
        pltpu.VMEM((ROWS_PER_TILE,), jnp.float32),
    ],
    compiler_params=_sc_params,
)
def _deg_kernel(colp_hbm, ewp_hbm, deg_hbm, deg_sh, col_v, ew_v, z_v):
    cid = lax.axis_index("c")
    sid = lax.axis_index("s")

    def zb(i, _):
        z_v[pl.ds(i * 16, 16)] = _z16()
        return 0
    lax.fori_loop(0, ROWS_PER_TILE // 16, zb, 0)
    pltpu.sync_copy(z_v, deg_sh.at[pl.ds(sid * ROWS_PER_TILE, ROWS_PER_TILE)])
    plsc.subcore_barrier()

    base = sid * DEG_CH
    pltpu.sync_copy(colp_hbm.at[pl.ds(base, DEG_CH)], col_v)
    pltpu.sync_copy(ewp_hbm.at[pl.ds(base, DEG_CH)], ew_v)

    def db(c, _):
        pltpu.sync_copy(ew_v.at[c], deg_sh.at[col_v.at[c]], add=True)
        return 0
    lax.fori_loop(0, DEG_CH, db, 0)
    plsc.subcore_barrier()

    @pl.when(cid == 0)
    def _():
        sl = pl.ds(sid * ROWS_PER_TILE, ROWS_PER_TILE)
        pltpu.sync_copy(deg_sh.at[sl], z_v)
        pltpu.sync_copy(z_v, deg_hbm.at[sl])


@functools.partial(
    pl.kernel,
    out_type=jax.ShapeDtypeStruct((NC, NPAD, D), jnp.float32),
    mesh=_mesh,
    scratch_types=[
        pltpu.VMEM_SHARED((NPAD, D), jnp.float32),
        pltpu.VMEM((ECH, 128), jnp.int32),
        pltpu.VMEM((ECH, 128), jnp.int32),
        pltpu.VMEM((ECH, 128), jnp.float32),
        pltpu.VMEM((128, D), jnp.float32),
        pltpu.VMEM((128,), jnp.float32),
    ],
    compiler_params=_sc_params,
)
def _agg_kernel(xwp_hbm, rowp_hbm, colp_hbm, ewp_hbm, part_hbm,
                acc_sh, rch, cch, ech, rows_v, en_v):
    cid = lax.axis_index("c")
    sid = lax.axis_index("s")
    wid = sid * NC + cid

    def zb(i, _):
        for d in range(D // 16):
            rows_v[i, pl.ds(d * 16, 16)] = _z16()
        return 0
    lax.fori_loop(0, 128, zb, 0)

    def zcopy(b, _):
        pltpu.sync_copy(
            rows_v, acc_sh.at[pl.ds(sid * ROWS_PER_TILE + b * 128, 128)])
        return 0
    lax.fori_loop(0, ROWS_PER_TILE // 128, zcopy, 0)
    plsc.subcore_barrier()

    eb = wid * ECH
    pltpu.sync_copy(rowp_hbm.at[pl.ds(eb, ECH)], rch)
    pltpu.sync_copy(colp_hbm.at[pl.ds(eb, ECH)], cch)
    pltpu.sync_copy(ewp_hbm.at[pl.ds(eb, ECH)], ech)

    def chunk(c, _):
        pltpu.sync_copy(xwp_hbm.at[rch.at[c]], rows_v)
        for d in range(128 // 16):
            en_v[pl.ds(d * 16, 16)] = ech[c, pl.ds(d * 16, 16)]

        def scale(k, _):
            k16 = jnp.full((16,), k, jnp.int32)
            w16 = plsc.load_gather(en_v, [k16])
            for d in range(D // 16):
                rows_v[k, pl.ds(d * 16, 16)] = rows_v[k, pl.ds(d * 16, 16)] * w16
            return 0
        lax.fori_loop(0, 128, scale, 0)

        pltpu.sync_copy(rows_v, acc_sh.at[cch.at[c]], add=True)
        return 0
    lax.fori_loop(0, ECH, chunk, 0)
    plsc.subcore_barrier()

    def wo(b, _):
        r0 = sid * ROWS_PER_TILE + b * 128
        pltpu.sync_copy(acc_sh.at[pl.ds(r0, 128)], rows_v)
        pltpu.sync_copy(rows_v, part_hbm.at[cid, pl.ds(r0, 128)])
        return 0
    lax.fori_loop(0, ROWS_PER_TILE // 128, wo, 0)


BM = 512
GRID = NPAD // BM


def _mm1_body(x_ref, w_ref, deg_ref, xwp_ref, dis_ref):
    dis = lax.rsqrt(deg_ref[...] + 1.0)
    xw = jnp.dot(x_ref[...], w_ref[...], preferred_element_type=jnp.float32)
    xwp_ref[...] = dis * xw
    dis_ref[...] = dis


_mm1 = pl.pallas_call(
    _mm1_body,
    grid=(GRID,),
    in_specs=[
        pl.BlockSpec((BM, D), lambda i: (i, 0)),
        pl.BlockSpec((D, D), lambda i: (0, 0)),
        pl.BlockSpec((BM, 1), lambda i: (i, 0)),
    ],
    out_specs=[
        pl.BlockSpec((BM, D), lambda i: (i, 0)),
        pl.BlockSpec((BM, 1), lambda i: (i, 0)),
    ],
    out_shape=[
        jax.ShapeDtypeStruct((NPAD, D), jnp.float32),
        jax.ShapeDtypeStruct((NPAD, 1), jnp.float32),
    ],
)


def _mm2_body(p0_ref, p1_ref, xwp_ref, dis_ref, b_ref, w_ref, out_ref):
    dis = dis_ref[...]
    s = dis * (p0_ref[...] + p1_ref[...] + xwp_ref[...]) + b_ref[...]
    h = jnp.maximum(s, 0.0)
    out_ref[...] = dis * jnp.dot(h, w_ref[...], preferred_element_type=jnp.float32)


_mm2 = pl.pallas_call(
    _mm2_body,
    grid=(GRID,),
    in_specs=[
        pl.BlockSpec((BM, D), lambda i: (i, 0)),
        pl.BlockSpec((BM, D), lambda i: (i, 0)),
        pl.BlockSpec((BM, D), lambda i: (i, 0)),
        pl.BlockSpec((BM, 1), lambda i: (i, 0)),
        pl.BlockSpec((1, D), lambda i: (0, 0)),
        pl.BlockSpec((D, D), lambda i: (0, 0)),
    ],
    out_specs=pl.BlockSpec((BM, D), lambda i: (i, 0)),
    out_shape=jax.ShapeDtypeStruct((NPAD, D), jnp.float32),
)


def _fin_body(q0_ref, q1_ref, xwp_ref, dis_ref, b_ref, out_ref):
    s = dis_ref[...] * (q0_ref[...] + q1_ref[...] + xwp_ref[...]) + b_ref[...]
    out_ref[...] = jax.nn.sigmoid(s)


_fin = pl.pallas_call(
    _fin_body,
    grid=(GRID,),
    in_specs=[
        pl.BlockSpec((BM, D), lambda i: (i, 0)),
        pl.BlockSpec((BM, D), lambda i: (i, 0)),
        pl.BlockSpec((BM, D), lambda i: (i, 0)),
        pl.BlockSpec((BM, 1), lambda i: (i, 0)),
        pl.BlockSpec((1, D), lambda i: (0, 0)),
    ],
    out_specs=pl.BlockSpec((BM, D), lambda i: (i, 0)),
    out_shape=jax.ShapeDtypeStruct((NPAD, D), jnp.float32),
)


def kernel(x, edge_index, edge_weight, W1, b1, W2, b2):
    row = edge_index[0].astype(jnp.int32)
    col = edge_index[1].astype(jnp.int32)
    ew = edge_weight.astype(jnp.float32)

    npad_e = EPAD - E
    pad_idx = jnp.arange(npad_e, dtype=jnp.int32) % N
    rowp = jnp.concatenate([row, pad_idx]).reshape(EPAD // 128, 128)
    colp = jnp.concatenate([col, pad_idx]).reshape(EPAD // 128, 128)
    ewp = jnp.concatenate([ew, jnp.zeros((npad_e,), jnp.float32)]
                          ).reshape(EPAD // 128, 128)
    xpad = jnp.concatenate(
        [x.astype(jnp.float32), jnp.zeros((NPAD - N, D), jnp.float32)])

    deg = _deg_kernel(colp, ewp)
    xwp1, dis = _mm1(xpad, W1, deg.reshape(NPAD, 1))
    p = _agg_kernel(xwp1, rowp, colp, ewp)
    xwp2 = _mm2(p[0], p[1], xwp1, dis, b1.reshape(1, D), W2)
    q = _agg_kernel(xwp2, rowp, colp, ewp)
    out = _fin(q[0], q[1], xwp2, dis, b2.reshape(1, D))
    return out[:N]


# speedup vs baseline: 17.1122x; 1.1909x over previous
import functools
import jax
import jax.numpy as jnp
from jax import lax
from jax.experimental import pallas as pl
from jax.experimental.pallas import tpu as pltpu
from jax.experimental.pallas import tpu_sc as plsc

N = 10000
E = 320000
D = 128

NC = 2
NS = 16
NW = NC * NS

NPAD = 10240
EPAD = 327680
ROWS_PER_TILE = NPAD // NS
ECH = EPAD // (NW * 128)
DEG_CH = EPAD // (NS * 128)

_mesh = plsc.VectorSubcoreMesh(
    core_axis_name="c", subcore_axis_name="s", num_cores=NC, num_subcores=NS)

_sc_params = pltpu.CompilerParams(needs_layout_passes=False)


def _z16():
    return jnp.zeros((16,), jnp.float32)


@functools.partial(
    pl.kernel,
    out_type=jax.ShapeDtypeStruct((NPAD,), jnp.float32),
    mesh=_mesh,
    scratch_types=[
        pltpu.VMEM_SHARED((NPAD,), jnp.float32),
        pltpu.VMEM((DEG_CH, 128), jnp.int32),
        pltpu.VMEM((DEG_CH, 128), jnp.float32),
        pltpu.VMEM((ROWS_PER_TILE,), jnp.float32),
    ],
    compiler_params=_sc_params,
)
def _deg_kernel(colp_hbm, ewp_hbm, deg_hbm, deg_sh, col_v, ew_v, z_v):
    cid = lax.axis_index("c")
    sid = lax.axis_index("s")

    def zb(i, _):
        z_v[pl.ds(i * 16, 16)] = _z16()
        return 0
    lax.fori_loop(0, ROWS_PER_TILE // 16, zb, 0)
    pltpu.sync_copy(z_v, deg_sh.at[pl.ds(sid * ROWS_PER_TILE, ROWS_PER_TILE)])
    plsc.subcore_barrier()

    base = sid * DEG_CH
    pltpu.sync_copy(colp_hbm.at[pl.ds(base, DEG_CH)], col_v)
    pltpu.sync_copy(ewp_hbm.at[pl.ds(base, DEG_CH)], ew_v)

    def db(c, _):
        pltpu.sync_copy(ew_v.at[c], deg_sh.at[col_v.at[c]], add=True)
        return 0
    lax.fori_loop(0, DEG_CH, db, 0)
    plsc.subcore_barrier()

    @pl.when(cid == 0)
    def _():
        sl = pl.ds(sid * ROWS_PER_TILE, ROWS_PER_TILE)
        pltpu.sync_copy(deg_sh.at[sl], z_v)
        pltpu.sync_copy(z_v, deg_hbm.at[sl])


_GDIMS = lax.GatherDimensionNumbers(
    offset_dims=(), collapsed_slice_dims=(0,), start_index_map=(0,))


def _bcast_lane(ev, j):
    return lax.gather(ev, jnp.full((16, 1), j, jnp.int32), _GDIMS,
                      slice_sizes=(1,),
                      mode=lax.GatherScatterMode.PROMISE_IN_BOUNDS)


@functools.partial(
    pl.kernel,
    out_type=jax.ShapeDtypeStruct((NC, NPAD, D), jnp.float32),
    mesh=_mesh,
    scratch_types=[
        pltpu.VMEM_SHARED((NPAD, D), jnp.float32),
        pltpu.VMEM((ECH, 128), jnp.int32),
        pltpu.VMEM((ECH, 128), jnp.int32),
        pltpu.VMEM((ECH, 128), jnp.float32),
        pltpu.VMEM((128, D), jnp.float32),
    ],
    compiler_params=_sc_params,
)
def _agg_kernel(xwp_hbm, rowp_hbm, colp_hbm, ewp_hbm, part_hbm,
                acc_sh, rch, cch, ech, rows_v):
    cid = lax.axis_index("c")
    sid = lax.axis_index("s")
    wid = sid * NC + cid

    def zb(i, _):
        for d in range(D // 16):
            rows_v[i, pl.ds(d * 16, 16)] = _z16()
        return 0
    lax.fori_loop(0, 128, zb, 0)

    def zcopy(b, _):
        pltpu.sync_copy(
            rows_v, acc_sh.at[pl.ds(sid * ROWS_PER_TILE + b * 128, 128)])
        return 0
    lax.fori_loop(0, ROWS_PER_TILE // 128, zcopy, 0)
    plsc.subcore_barrier()

    eb = wid * ECH
    pltpu.sync_copy(rowp_hbm.at[pl.ds(eb, ECH)], rch)
    pltpu.sync_copy(colp_hbm.at[pl.ds(eb, ECH)], cch)
    pltpu.sync_copy(ewp_hbm.at[pl.ds(eb, ECH)], ech)

    def chunk(c, _):
        pltpu.sync_copy(xwp_hbm.at[rch.at[c]], rows_v)

        def grp(g, _):
            ev = ech[c, pl.ds(g * 16, 16)]
            for j in range(16):
                w16 = _bcast_lane(ev, j)
                k = g * 16 + j
                for d in range(D // 16):
                    rows_v[k, pl.ds(d * 16, 16)] = (
                        rows_v[k, pl.ds(d * 16, 16)] * w16)
            return 0
        lax.fori_loop(0, 8, grp, 0)

        pltpu.sync_copy(rows_v, acc_sh.at[cch.at[c]], add=True)
        return 0
    lax.fori_loop(0, ECH, chunk, 0)
    plsc.subcore_barrier()

    def wo(b, _):
        r0 = sid * ROWS_PER_TILE + b * 128
        pltpu.sync_copy(acc_sh.at[pl.ds(r0, 128)], rows_v)
        pltpu.sync_copy(rows_v, part_hbm.at[cid, pl.ds(r0, 128)])
        return 0
    lax.fori_loop(0, ROWS_PER_TILE // 128, wo, 0)


BM = 512
GRID = NPAD // BM


def _mm1_body(x_ref, w_ref, deg_ref, xwp_ref, dis_ref):
    dis = lax.rsqrt(deg_ref[...] + 1.0)
    xw = jnp.dot(x_ref[...], w_ref[...], preferred_element_type=jnp.float32)
    xwp_ref[...] = dis * xw
    dis_ref[...] = dis


_mm1 = pl.pallas_call(
    _mm1_body,
    grid=(GRID,),
    in_specs=[
        pl.BlockSpec((BM, D), lambda i: (i, 0)),
        pl.BlockSpec((D, D), lambda i: (0, 0)),
        pl.BlockSpec((BM, 1), lambda i: (i, 0)),
    ],
    out_specs=[
        pl.BlockSpec((BM, D), lambda i: (i, 0)),
        pl.BlockSpec((BM, 1), lambda i: (i, 0)),
    ],
    out_shape=[
        jax.ShapeDtypeStruct((NPAD, D), jnp.float32),
        jax.ShapeDtypeStruct((NPAD, 1), jnp.float32),
    ],
)


def _mm2_body(p0_ref, p1_ref, xwp_ref, dis_ref, b_ref, w_ref, out_ref):
    dis = dis_ref[...]
    s = dis * (p0_ref[...] + p1_ref[...] + xwp_ref[...]) + b_ref[...]
    h = jnp.maximum(s, 0.0)
    out_ref[...] = dis * jnp.dot(h, w_ref[...], preferred_element_type=jnp.float32)


_mm2 = pl.pallas_call(
    _mm2_body,
    grid=(GRID,),
    in_specs=[
        pl.BlockSpec((BM, D), lambda i: (i, 0)),
        pl.BlockSpec((BM, D), lambda i: (i, 0)),
        pl.BlockSpec((BM, D), lambda i: (i, 0)),
        pl.BlockSpec((BM, 1), lambda i: (i, 0)),
        pl.BlockSpec((1, D), lambda i: (0, 0)),
        pl.BlockSpec((D, D), lambda i: (0, 0)),
    ],
    out_specs=pl.BlockSpec((BM, D), lambda i: (i, 0)),
    out_shape=jax.ShapeDtypeStruct((NPAD, D), jnp.float32),
)


def _fin_body(q0_ref, q1_ref, xwp_ref, dis_ref, b_ref, out_ref):
    s = dis_ref[...] * (q0_ref[...] + q1_ref[...] + xwp_ref[...]) + b_ref[...]
    out_ref[...] = jax.nn.sigmoid(s)


_fin = pl.pallas_call(
    _fin_body,
    grid=(GRID,),
    in_specs=[
        pl.BlockSpec((BM, D), lambda i: (i, 0)),
        pl.BlockSpec((BM, D), lambda i: (i, 0)),
        pl.BlockSpec((BM, D), lambda i: (i, 0)),
        pl.BlockSpec((BM, 1), lambda i: (i, 0)),
        pl.BlockSpec((1, D), lambda i: (0, 0)),
    ],
    out_specs=pl.BlockSpec((BM, D), lambda i: (i, 0)),
    out_shape=jax.ShapeDtypeStruct((NPAD, D), jnp.float32),
)


def kernel(x, edge_index, edge_weight, W1, b1, W2, b2):
    row = edge_index[0].astype(jnp.int32)
    col = edge_index[1].astype(jnp.int32)
    ew = edge_weight.astype(jnp.float32)

    npad_e = EPAD - E
    pad_idx = jnp.arange(npad_e, dtype=jnp.int32) % N
    rowp = jnp.concatenate([row, pad_idx]).reshape(EPAD // 128, 128)
    colp = jnp.concatenate([col, pad_idx]).reshape(EPAD // 128, 128)
    ewp = jnp.concatenate([ew, jnp.zeros((npad_e,), jnp.float32)]
                          ).reshape(EPAD // 128, 128)
    xpad = jnp.concatenate(
        [x.astype(jnp.float32), jnp.zeros((NPAD - N, D), jnp.float32)])

    deg = _deg_kernel(colp, ewp)
    xwp1, dis = _mm1(xpad, W1, deg.reshape(NPAD, 1))
    p = _agg_kernel(xwp1, rowp, colp, ewp)
    xwp2 = _mm2(p[0], p[1], xwp1, dis, b1.reshape(1, D), W2)
    q = _agg_kernel(xwp2, rowp, colp, ewp)
    out = _fin(q[0], q[1], xwp2, dis, b2.reshape(1, D))
    return out[:N]


# speedup vs baseline: 23.6767x; 1.3836x over previous
import functools
import jax
import jax.numpy as jnp
from jax import lax
from jax.experimental import pallas as pl
from jax.experimental.pallas import tpu as pltpu
from jax.experimental.pallas import tpu_sc as plsc

N = 10000
E = 320000
D = 128

NC = 2
NS = 16
NW = NC * NS

NPAD = 10240
EPAD = 327680
ROWS_PER_TILE = NPAD // NS
ECH = EPAD // (NW * 128)
DEG_CH = EPAD // (NS * 128)

_mesh = plsc.VectorSubcoreMesh(
    core_axis_name="c", subcore_axis_name="s", num_cores=NC, num_subcores=NS)

_sc_params = pltpu.CompilerParams(needs_layout_passes=False)


def _z16():
    return jnp.zeros((16,), jnp.float32)


@functools.partial(
    pl.kernel,
    out_type=jax.ShapeDtypeStruct((NPAD,), jnp.float32),
    mesh=_mesh,
    scratch_types=[
        pltpu.VMEM_SHARED((NPAD,), jnp.float32),
        pltpu.VMEM((DEG_CH, 128), jnp.int32),
        pltpu.VMEM((DEG_CH, 128), jnp.float32),
        pltpu.VMEM((ROWS_PER_TILE,), jnp.float32),
    ],
    compiler_params=_sc_params,
)
def _deg_kernel(colp_hbm, ewp_hbm, deg_hbm, deg_sh, col_v, ew_v, z_v):
    cid = lax.axis_index("c")
    sid = lax.axis_index("s")

    def zb(i, _):
        z_v[pl.ds(i * 16, 16)] = _z16()
        return 0
    lax.fori_loop(0, ROWS_PER_TILE // 16, zb, 0)
    pltpu.sync_copy(z_v, deg_sh.at[pl.ds(sid * ROWS_PER_TILE, ROWS_PER_TILE)])
    plsc.subcore_barrier()

    base = sid * DEG_CH
    pltpu.sync_copy(colp_hbm.at[pl.ds(base, DEG_CH)], col_v)
    pltpu.sync_copy(ewp_hbm.at[pl.ds(base, DEG_CH)], ew_v)

    def db(c, _):
        pltpu.sync_copy(ew_v.at[c], deg_sh.at[col_v.at[c]], add=True)
        return 0
    lax.fori_loop(0, DEG_CH, db, 0)
    plsc.subcore_barrier()

    @pl.when(cid == 0)
    def _():
        sl = pl.ds(sid * ROWS_PER_TILE, ROWS_PER_TILE)
        pltpu.sync_copy(deg_sh.at[sl], z_v)
        pltpu.sync_copy(z_v, deg_hbm.at[sl])


_GDIMS = lax.GatherDimensionNumbers(
    offset_dims=(), collapsed_slice_dims=(0,), start_index_map=(0,))


def _bcast_lane(ev, j):
    return lax.gather(ev, jnp.full((16, 1), j, jnp.int32), _GDIMS,
                      slice_sizes=(1,),
                      mode=lax.GatherScatterMode.PROMISE_IN_BOUNDS)


NBUF = 4
CK = 32
NCHUNK = EPAD // (NW * CK)


@functools.partial(
    pl.kernel,
    out_type=jax.ShapeDtypeStruct((NC, NPAD, D), jnp.float32),
    mesh=_mesh,
    scratch_types=(
        [
            pltpu.VMEM_SHARED((NPAD, D), jnp.float32),
            pltpu.VMEM((ECH, 128), jnp.int32),
            pltpu.VMEM((ECH, 128), jnp.int32),
            pltpu.VMEM((ECH, 128), jnp.float32),
            pltpu.VMEM((NBUF, CK), jnp.int32),
            pltpu.VMEM((NBUF, CK), jnp.int32),
        ]
        + [pltpu.VMEM((CK, D), jnp.float32)] * NBUF
        + [pltpu.SemaphoreType.DMA] * (2 * NBUF)
    ),
    compiler_params=_sc_params,
)
def _agg_kernel(xwp_hbm, rowp_hbm, colp_hbm, ewp_hbm, part_hbm,
                acc_sh, rch, cch, ech, ridx, cidx, *rest):
    rows = rest[:NBUF]
    gsem = rest[NBUF:2 * NBUF]
    ssem = rest[2 * NBUF:]
    cid = lax.axis_index("c")
    sid = lax.axis_index("s")
    wid = sid * NC + cid

    eb = wid * ECH
    pltpu.sync_copy(rowp_hbm.at[pl.ds(eb, ECH)], rch)
    pltpu.sync_copy(colp_hbm.at[pl.ds(eb, ECH)], cch)
    pltpu.sync_copy(ewp_hbm.at[pl.ds(eb, ECH)], ech)

    def zb(i, _):
        for d in range(D // 16):
            rows[0][i, pl.ds(d * 16, 16)] = _z16()
        return 0
    lax.fori_loop(0, CK, zb, 0)

    def zcopy(b, _):
        pltpu.sync_copy(
            rows[0], acc_sh.at[pl.ds(sid * ROWS_PER_TILE + b * CK, CK)])
        return 0
    lax.fori_loop(0, ROWS_PER_TILE // CK, zcopy, 0)
    plsc.subcore_barrier()

    def _copy_idx(src, dst_ring, slot, r128, sub):
        for g in range(CK // 16):
            dst_ring[slot, pl.ds(g * 16, 16)] = (
                src[r128, pl.ds(sub * CK + g * 16, 16)])

    for p in range(NBUF - 1):
        _copy_idx(rch, ridx, p, 0, p)
        pltpu.async_copy(xwp_hbm.at[ridx.at[p]], rows[p], gsem[p])

    def outer(o, _):
        for b in range(NBUF):
            nb = (b + NBUF - 1) % NBUF
            nr = o + (b + NBUF - 1) // NBUF
            nsub = (b + NBUF - 1) % NBUF

            if b == 0:
                @pl.when(o > 0)
                def _():
                    pltpu.make_async_copy(
                        rows[nb], acc_sh.at[cidx.at[nb]], ssem[nb]).wait()
            else:
                pltpu.make_async_copy(
                    rows[nb], acc_sh.at[cidx.at[nb]], ssem[nb]).wait()

            @pl.when(4 * o + b + NBUF - 1 < NCHUNK)
            def _():
                _copy_idx(rch, ridx, nb, nr, nsub)
                pltpu.async_copy(xwp_hbm.at[ridx.at[nb]], rows[nb], gsem[nb])

            pltpu.make_async_copy(
                xwp_hbm.at[ridx.at[b]], rows[b], gsem[b]).wait()

            for g in range(CK // 16):
                ev = ech[o, pl.ds(b * CK + g * 16, 16)]
                for j in range(16):
                    w16 = _bcast_lane(ev, j)
                    k = g * 16 + j
                    for d in range(D // 16):
                        rows[b][k, pl.ds(d * 16, 16)] = (
                            rows[b][k, pl.ds(d * 16, 16)] * w16)

            _copy_idx(cch, cidx, b, o, b)
            pltpu.async_copy(rows[b], acc_sh.at[cidx.at[b]], ssem[b], add=True)
        return 0
    lax.fori_loop(0, NCHUNK // NBUF, outer, 0)

    pltpu.make_async_copy(
        rows[NBUF - 1], acc_sh.at[cidx.at[NBUF - 1]], ssem[NBUF - 1]).wait()
    plsc.subcore_barrier()

    def wo(b, _):
        r0 = sid * ROWS_PER_TILE + b * CK
        pltpu.sync_copy(acc_sh.at[pl.ds(r0, CK)], rows[0])
        pltpu.sync_copy(rows[0], part_hbm.at[cid, pl.ds(r0, CK)])
        return 0
    lax.fori_loop(0, ROWS_PER_TILE // CK, wo, 0)


BM = 512
GRID = NPAD // BM


def _mm1_body(x_ref, w_ref, deg_ref, xwp_ref, dis_ref):
    dis = lax.rsqrt(deg_ref[...] + 1.0)
    xw = jnp.dot(x_ref[...], w_ref[...], preferred_element_type=jnp.float32)
    xwp_ref[...] = dis * xw
    dis_ref[...] = dis


_mm1 = pl.pallas_call(
    _mm1_body,
    grid=(GRID,),
    in_specs=[
        pl.BlockSpec((BM, D), lambda i: (i, 0)),
        pl.BlockSpec((D, D), lambda i: (0, 0)),
        pl.BlockSpec((BM, 1), lambda i: (i, 0)),
    ],
    out_specs=[
        pl.BlockSpec((BM, D), lambda i: (i, 0)),
        pl.BlockSpec((BM, 1), lambda i: (i, 0)),
    ],
    out_shape=[
        jax.ShapeDtypeStruct((NPAD, D), jnp.float32),
        jax.ShapeDtypeStruct((NPAD, 1), jnp.float32),
    ],
)


def _mm2_body(p0_ref, p1_ref, xwp_ref, dis_ref, b_ref, w_ref, out_ref):
    dis = dis_ref[...]
    s = dis * (p0_ref[...] + p1_ref[...] + xwp_ref[...]) + b_ref[...]
    h = jnp.maximum(s, 0.0)
    out_ref[...] = dis * jnp.dot(h, w_ref[...], preferred_element_type=jnp.float32)


_mm2 = pl.pallas_call(
    _mm2_body,
    grid=(GRID,),
    in_specs=[
        pl.BlockSpec((BM, D), lambda i: (i, 0)),
        pl.BlockSpec((BM, D), lambda i: (i, 0)),
        pl.BlockSpec((BM, D), lambda i: (i, 0)),
        pl.BlockSpec((BM, 1), lambda i: (i, 0)),
        pl.BlockSpec((1, D), lambda i: (0, 0)),
        pl.BlockSpec((D, D), lambda i: (0, 0)),
    ],
    out_specs=pl.BlockSpec((BM, D), lambda i: (i, 0)),
    out_shape=jax.ShapeDtypeStruct((NPAD, D), jnp.float32),
)


def _fin_body(q0_ref, q1_ref, xwp_ref, dis_ref, b_ref, out_ref):
    s = dis_ref[...] * (q0_ref[...] + q1_ref[...] + xwp_ref[...]) + b_ref[...]
    out_ref[...] = jax.nn.sigmoid(s)


_fin = pl.pallas_call(
    _fin_body,
    grid=(GRID,),
    in_specs=[
        pl.BlockSpec((BM, D), lambda i: (i, 0)),
        pl.BlockSpec((BM, D), lambda i: (i, 0)),
        pl.BlockSpec((BM, D), lambda i: (i, 0)),
        pl.BlockSpec((BM, 1), lambda i: (i, 0)),
        pl.BlockSpec((1, D), lambda i: (0, 0)),
    ],
    out_specs=pl.BlockSpec((BM, D), lambda i: (i, 0)),
    out_shape=jax.ShapeDtypeStruct((NPAD, D), jnp.float32),
)


def kernel(x, edge_index, edge_weight, W1, b1, W2, b2):
    row = edge_index[0].astype(jnp.int32)
    col = edge_index[1].astype(jnp.int32)
    ew = edge_weight.astype(jnp.float32)

    npad_e = EPAD - E
    pad_idx = jnp.arange(npad_e, dtype=jnp.int32) % N
    rowp = jnp.concatenate([row, pad_idx]).reshape(EPAD // 128, 128)
    colp = jnp.concatenate([col, pad_idx]).reshape(EPAD // 128, 128)
    ewp = jnp.concatenate([ew, jnp.zeros((npad_e,), jnp.float32)]
                          ).reshape(EPAD // 128, 128)
    xpad = jnp.concatenate(
        [x.astype(jnp.float32), jnp.zeros((NPAD - N, D), jnp.float32)])

    deg = _deg_kernel(colp, ewp)
    xwp1, dis = _mm1(xpad, W1, deg.reshape(NPAD, 1))
    p = _agg_kernel(xwp1, rowp, colp, ewp)
    xwp2 = _mm2(p[0], p[1], xwp1, dis, b1.reshape(1, D), W2)
    q = _agg_kernel(xwp2, rowp, colp, ewp)
    out = _fin(q[0], q[1], xwp2, dis, b2.reshape(1, D))
    return out[:N]


# speedup vs baseline: 25.7052x; 1.0857x over previous
import functools
import jax
import jax.numpy as jnp
from jax import lax
from jax.experimental import pallas as pl
from jax.experimental.pallas import tpu as pltpu
from jax.experimental.pallas import tpu_sc as plsc

N = 10000
E = 320000
D = 128

NC = 2
NS = 16
NW = NC * NS

NPAD = 10240
EPAD = 327680
ROWS_PER_TILE = NPAD // NS
ECH = EPAD // (NW * 128)
DEG_CH = EPAD // (NS * 128)

_mesh = plsc.VectorSubcoreMesh(
    core_axis_name="c", subcore_axis_name="s", num_cores=NC, num_subcores=NS)

_sc_params = pltpu.CompilerParams(needs_layout_passes=False)


def _z16():
    return jnp.zeros((16,), jnp.float32)


@functools.partial(
    pl.kernel,
    out_type=jax.ShapeDtypeStruct((NPAD,), jnp.float32),
    mesh=_mesh,
    scratch_types=[
        pltpu.VMEM_SHARED((NPAD,), jnp.float32),
        pltpu.VMEM((DEG_CH, 128), jnp.int32),
        pltpu.VMEM((DEG_CH, 128), jnp.float32),
        pltpu.VMEM((ROWS_PER_TILE,), jnp.float32),
    ],
    compiler_params=_sc_params,
)
def _deg_kernel(colp_hbm, ewp_hbm, deg_hbm, deg_sh, col_v, ew_v, z_v):
    cid = lax.axis_index("c")
    sid = lax.axis_index("s")

    def zb(i, _):
        z_v[pl.ds(i * 16, 16)] = _z16()
        return 0
    lax.fori_loop(0, ROWS_PER_TILE // 16, zb, 0)
    pltpu.sync_copy(z_v, deg_sh.at[pl.ds(sid * ROWS_PER_TILE, ROWS_PER_TILE)])
    plsc.subcore_barrier()

    base = sid * DEG_CH
    pltpu.sync_copy(colp_hbm.at[pl.ds(base, DEG_CH)], col_v)
    pltpu.sync_copy(ewp_hbm.at[pl.ds(base, DEG_CH)], ew_v)

    def db(c, _):
        pltpu.sync_copy(ew_v.at[c], deg_sh.at[col_v.at[c]], add=True)
        return 0
    lax.fori_loop(0, DEG_CH, db, 0)
    plsc.subcore_barrier()

    @pl.when(cid == 0)
    def _():
        sl = pl.ds(sid * ROWS_PER_TILE, ROWS_PER_TILE)
        pltpu.sync_copy(deg_sh.at[sl], z_v)
        pltpu.sync_copy(z_v, deg_hbm.at[sl])


_GDIMS = lax.GatherDimensionNumbers(
    offset_dims=(), collapsed_slice_dims=(0,), start_index_map=(0,))


def _bcast_lane(ev, j):
    return lax.gather(ev, jnp.full((16, 1), j, jnp.int32), _GDIMS,
                      slice_sizes=(1,),
                      mode=lax.GatherScatterMode.PROMISE_IN_BOUNDS)


NBUF = 4
CK = 64
NCHUNK = EPAD // (NW * CK)
SUBS = 128 // CK


@functools.partial(
    pl.kernel,
    out_type=jax.ShapeDtypeStruct((NC, NPAD, D), jnp.float32),
    mesh=_mesh,
    scratch_types=(
        [
            pltpu.VMEM_SHARED((NPAD, D), jnp.float32),
            pltpu.VMEM((ECH, 128), jnp.int32),
            pltpu.VMEM((NBUF, CK), jnp.int32),
            pltpu.VMEM((NBUF, CK), jnp.int32),
            pltpu.VMEM((NBUF, CK), jnp.float32),
        ]
        + [pltpu.VMEM((CK, D), jnp.float32)] * NBUF
        + [pltpu.SemaphoreType.DMA] * (3 * NBUF)
    ),
    compiler_params=_sc_params,
)
def _agg_kernel(xwp_hbm, rcp_hbm, ewp_hbm, part_hbm,
                acc_sh, rcv, ridx, cidx, ewr, *rest):
    rows = rest[:NBUF]
    gsem = rest[NBUF:2 * NBUF]
    ssem = rest[2 * NBUF:3 * NBUF]
    esem = rest[3 * NBUF:]
    cid = lax.axis_index("c")
    sid = lax.axis_index("s")
    wid = sid * NC + cid

    eb = wid * ECH
    pltpu.sync_copy(rcp_hbm.at[pl.ds(eb, ECH)], rcv)

    def zb(i, _):
        for d in range(D // 16):
            rows[0][i, pl.ds(d * 16, 16)] = _z16()
        return 0
    lax.fori_loop(0, CK, zb, 0)

    def zcopy(b, _):
        pltpu.sync_copy(
            rows[0], acc_sh.at[pl.ds(sid * ROWS_PER_TILE + b * CK, CK)])
        return 0
    lax.fori_loop(0, ROWS_PER_TILE // CK, zcopy, 0)
    plsc.subcore_barrier()

    def _prep_chunk(slot, r128, sub):
        for g in range(CK // 16):
            v = rcv[r128, pl.ds(sub * CK + g * 16, 16)]
            ridx[slot, pl.ds(g * 16, 16)] = lax.shift_right_logical(v, 14)
            cidx[slot, pl.ds(g * 16, 16)] = lax.bitwise_and(v, 16383)
        pltpu.async_copy(
            ewp_hbm.at[r128 + eb, pl.ds(sub * CK, CK)], ewr.at[slot],
            esem[slot])
        pltpu.async_copy(xwp_hbm.at[ridx.at[slot]], rows[slot], gsem[slot])

    for p in range(NBUF - 1):
        _prep_chunk(p, p // SUBS, p % SUBS)

    def outer(o, _):
        for b in range(NBUF):
            nb = (b + NBUF - 1) % NBUF
            nr = SUBS * o + (b + NBUF - 1) // SUBS
            nsub = (b + NBUF - 1) % SUBS

            if b == 0:
                @pl.when(o > 0)
                def _():
                    pltpu.make_async_copy(
                        rows[nb], acc_sh.at[cidx.at[nb]], ssem[nb]).wait()
            else:
                pltpu.make_async_copy(
                    rows[nb], acc_sh.at[cidx.at[nb]], ssem[nb]).wait()

            @pl.when(NBUF * o + b + NBUF - 1 < NCHUNK)
            def _():
                _prep_chunk(nb, nr, nsub)

            pltpu.make_async_copy(
                xwp_hbm.at[ridx.at[b]], rows[b], gsem[b]).wait()
            pltpu.make_async_copy(
                ewp_hbm.at[0, pl.ds(0, CK)], ewr.at[b], esem[b]).wait()

            def grp(g, _):
                ev = ewr[b, pl.ds(g * 16, 16)]
                for j in range(16):
                    w16 = _bcast_lane(ev, j)
                    k = g * 16 + j
                    for d in range(D // 16):
                        rows[b][k, pl.ds(d * 16, 16)] = (
                            rows[b][k, pl.ds(d * 16, 16)] * w16)
                return 0
            lax.fori_loop(0, CK // 16, grp, 0)

            pltpu.async_copy(rows[b], acc_sh.at[cidx.at[b]], ssem[b], add=True)
        return 0
    lax.fori_loop(0, NCHUNK // NBUF, outer, 0)

    pltpu.make_async_copy(
        rows[NBUF - 1], acc_sh.at[cidx.at[NBUF - 1]], ssem[NBUF - 1]).wait()
    plsc.subcore_barrier()

    def wo(b, _):
        r0 = sid * ROWS_PER_TILE + b * CK
        pltpu.sync_copy(acc_sh.at[pl.ds(r0, CK)], rows[0])
        pltpu.sync_copy(rows[0], part_hbm.at[cid, pl.ds(r0, CK)])
        return 0
    lax.fori_loop(0, ROWS_PER_TILE // CK, wo, 0)


BM = 512
GRID = NPAD // BM


def _mm1_body(x_ref, w_ref, deg_ref, xwp_ref, dis_ref):
    dis = lax.rsqrt(deg_ref[...] + 1.0)
    xw = jnp.dot(x_ref[...], w_ref[...], preferred_element_type=jnp.float32)
    xwp_ref[...] = dis * xw
    dis_ref[...] = dis


_mm1 = pl.pallas_call(
    _mm1_body,
    grid=(GRID,),
    in_specs=[
        pl.BlockSpec((BM, D), lambda i: (i, 0)),
        pl.BlockSpec((D, D), lambda i: (0, 0)),
        pl.BlockSpec((BM, 1), lambda i: (i, 0)),
    ],
    out_specs=[
        pl.BlockSpec((BM, D), lambda i: (i, 0)),
        pl.BlockSpec((BM, 1), lambda i: (i, 0)),
    ],
    out_shape=[
        jax.ShapeDtypeStruct((NPAD, D), jnp.float32),
        jax.ShapeDtypeStruct((NPAD, 1), jnp.float32),
    ],
)


def _mm2_body(p0_ref, p1_ref, xwp_ref, dis_ref, b_ref, w_ref, out_ref):
    dis = dis_ref[...]
    s = dis * (p0_ref[...] + p1_ref[...] + xwp_ref[...]) + b_ref[...]
    h = jnp.maximum(s, 0.0)
    out_ref[...] = dis * jnp.dot(h, w_ref[...], preferred_element_type=jnp.float32)


_mm2 = pl.pallas_call(
    _mm2_body,
    grid=(GRID,),
    in_specs=[
        pl.BlockSpec((BM, D), lambda i: (i, 0)),
        pl.BlockSpec((BM, D), lambda i: (i, 0)),
        pl.BlockSpec((BM, D), lambda i: (i, 0)),
        pl.BlockSpec((BM, 1), lambda i: (i, 0)),
        pl.BlockSpec((1, D), lambda i: (0, 0)),
        pl.BlockSpec((D, D), lambda i: (0, 0)),
    ],
    out_specs=pl.BlockSpec((BM, D), lambda i: (i, 0)),
    out_shape=jax.ShapeDtypeStruct((NPAD, D), jnp.float32),
)


def _fin_body(q0_ref, q1_ref, xwp_ref, dis_ref, b_ref, out_ref):
    s = dis_ref[...] * (q0_ref[...] + q1_ref[...] + xwp_ref[...]) + b_ref[...]
    out_ref[...] = jax.nn.sigmoid(s)


_fin = pl.pallas_call(
    _fin_body,
    grid=(GRID,),
    in_specs=[
        pl.BlockSpec((BM, D), lambda i: (i, 0)),
        pl.BlockSpec((BM, D), lambda i: (i, 0)),
        pl.BlockSpec((BM, D), lambda i: (i, 0)),
        pl.BlockSpec((BM, 1), lambda i: (i, 0)),
        pl.BlockSpec((1, D), lambda i: (0, 0)),
    ],
    out_specs=pl.BlockSpec((BM, D), lambda i: (i, 0)),
    out_shape=jax.ShapeDtypeStruct((NPAD, D), jnp.float32),
)


def kernel(x, edge_index, edge_weight, W1, b1, W2, b2):
    row = edge_index[0].astype(jnp.int32)
    col = edge_index[1].astype(jnp.int32)
    ew = edge_weight.astype(jnp.float32)

    npad_e = EPAD - E
    pad_idx = jnp.arange(npad_e, dtype=jnp.int32) % N
    rowf = jnp.concatenate([row, pad_idx])
    colf = jnp.concatenate([col, pad_idx])
    colp = colf.reshape(EPAD // 128, 128)
    rcp = ((rowf << 14) | colf).reshape(EPAD // 128, 128)
    ewp = jnp.concatenate([ew, jnp.zeros((npad_e,), jnp.float32)]
                          ).reshape(EPAD // 128, 128)
    xpad = jnp.concatenate(
        [x.astype(jnp.float32), jnp.zeros((NPAD - N, D), jnp.float32)])

    deg = _deg_kernel(colp, ewp)
    xwp1, dis = _mm1(xpad, W1, deg.reshape(NPAD, 1))
    p = _agg_kernel(xwp1, rcp, ewp)
    xwp2 = _mm2(p[0], p[1], xwp1, dis, b1.reshape(1, D), W2)
    q = _agg_kernel(xwp2, rcp, ewp)
    out = _fin(q[0], q[1], xwp2, dis, b2.reshape(1, D))
    return out[:N]


# speedup vs baseline: 26.1053x; 1.0156x over previous
import functools
import jax
import jax.numpy as jnp
from jax import lax
from jax.experimental import pallas as pl
from jax.experimental.pallas import tpu as pltpu
from jax.experimental.pallas import tpu_sc as plsc

N = 10000
E = 320000
D = 128

NC = 2
NS = 16
NW = NC * NS

NPAD = 10240
EPAD = 327680
ROWS_PER_TILE = NPAD // NS
ECH = EPAD // (NW * 128)
DEG_CH = EPAD // (NS * 128)

_mesh = plsc.VectorSubcoreMesh(
    core_axis_name="c", subcore_axis_name="s", num_cores=NC, num_subcores=NS)

_sc_params = pltpu.CompilerParams(needs_layout_passes=False)


def _z16():
    return jnp.zeros((16,), jnp.float32)


@functools.partial(
    pl.kernel,
    out_type=jax.ShapeDtypeStruct((NPAD,), jnp.float32),
    mesh=_mesh,
    scratch_types=[
        pltpu.VMEM_SHARED((NPAD,), jnp.float32),
        pltpu.VMEM((DEG_CH, 128), jnp.int32),
        pltpu.VMEM((DEG_CH, 128), jnp.float32),
        pltpu.VMEM((ROWS_PER_TILE,), jnp.float32),
        pltpu.SemaphoreType.DMA,
    ],
    compiler_params=_sc_params,
)
def _deg_kernel(colp_hbm, ewp_hbm, deg_hbm, deg_sh, col_v, ew_v, z_v, dsem):
    cid = lax.axis_index("c")
    sid = lax.axis_index("s")

    def zb(i, _):
        z_v[pl.ds(i * 16, 16)] = _z16()
        return 0
    lax.fori_loop(0, ROWS_PER_TILE // 16, zb, 0)
    pltpu.sync_copy(z_v, deg_sh.at[pl.ds(sid * ROWS_PER_TILE, ROWS_PER_TILE)])
    plsc.subcore_barrier()

    base = sid * DEG_CH
    pltpu.sync_copy(colp_hbm.at[pl.ds(base, DEG_CH)], col_v)
    pltpu.sync_copy(ewp_hbm.at[pl.ds(base, DEG_CH)], ew_v)

    def db(o, _):
        for b in range(16):
            c = o * 16 + b
            pltpu.async_copy(ew_v.at[c], deg_sh.at[col_v.at[c]], dsem,
                             add=True)
        for b in range(16):
            pltpu.make_async_copy(
                ew_v.at[0], deg_sh.at[col_v.at[0]], dsem).wait()
        return 0
    lax.fori_loop(0, DEG_CH // 16, db, 0)
    plsc.subcore_barrier()

    @pl.when(cid == 0)
    def _():
        sl = pl.ds(sid * ROWS_PER_TILE, ROWS_PER_TILE)
        pltpu.sync_copy(deg_sh.at[sl], z_v)
        pltpu.sync_copy(z_v, deg_hbm.at[sl])


_GDIMS = lax.GatherDimensionNumbers(
    offset_dims=(), collapsed_slice_dims=(0,), start_index_map=(0,))


def _bcast_lane(ev, j):
    return lax.gather(ev, jnp.full((16, 1), j, jnp.int32), _GDIMS,
                      slice_sizes=(1,),
                      mode=lax.GatherScatterMode.PROMISE_IN_BOUNDS)


NBUF = 4
CK = 64
NCHUNK = EPAD // (NW * CK)
SUBS = 128 // CK


@functools.partial(
    pl.kernel,
    out_type=jax.ShapeDtypeStruct((NC, NPAD, D), jnp.float32),
    mesh=_mesh,
    scratch_types=(
        [
            pltpu.VMEM_SHARED((NPAD, D), jnp.float32),
            pltpu.VMEM((ECH, 128), jnp.int32),
            pltpu.VMEM((NBUF, CK), jnp.int32),
            pltpu.VMEM((NBUF, CK), jnp.int32),
            pltpu.VMEM((NBUF, CK), jnp.float32),
        ]
        + [pltpu.VMEM((CK, D), jnp.float32)] * NBUF
        + [pltpu.SemaphoreType.DMA] * (3 * NBUF)
    ),
    compiler_params=_sc_params,
)
def _agg_kernel(xwp_hbm, rcp_hbm, ewp_hbm, part_hbm,
                acc_sh, rcv, ridx, cidx, ewr, *rest):
    rows = rest[:NBUF]
    gsem = rest[NBUF:2 * NBUF]
    ssem = rest[2 * NBUF:3 * NBUF]
    esem = rest[3 * NBUF:]
    cid = lax.axis_index("c")
    sid = lax.axis_index("s")
    wid = sid * NC + cid

    eb = wid * ECH
    pltpu.sync_copy(rcp_hbm.at[pl.ds(eb, ECH)], rcv)

    def zb(i, _):
        for d in range(D // 16):
            rows[0][i, pl.ds(d * 16, 16)] = _z16()
        return 0
    lax.fori_loop(0, CK, zb, 0)

    def zcopy(b, _):
        pltpu.sync_copy(
            rows[0], acc_sh.at[pl.ds(sid * ROWS_PER_TILE + b * CK, CK)])
        return 0
    lax.fori_loop(0, ROWS_PER_TILE // CK, zcopy, 0)
    plsc.subcore_barrier()

    def _prep_chunk(slot, r128, sub):
        for g in range(CK // 16):
            v = rcv[r128, pl.ds(sub * CK + g * 16, 16)]
            ridx[slot, pl.ds(g * 16, 16)] = lax.shift_right_logical(v, 14)
            cidx[slot, pl.ds(g * 16, 16)] = lax.bitwise_and(v, 16383)
        pltpu.async_copy(
            ewp_hbm.at[r128 + eb, pl.ds(sub * CK, CK)], ewr.at[slot],
            esem[slot])
        pltpu.async_copy(xwp_hbm.at[ridx.at[slot]], rows[slot], gsem[slot])

    for p in range(NBUF - 1):
        _prep_chunk(p, p // SUBS, p % SUBS)

    def outer(o, _):
        for b in range(NBUF):
            nb = (b + NBUF - 1) % NBUF
            nr = SUBS * o + (b + NBUF - 1) // SUBS
            nsub = (b + NBUF - 1) % SUBS

            if b == 0:
                @pl.when(o > 0)
                def _():
                    pltpu.make_async_copy(
                        rows[nb], acc_sh.at[cidx.at[nb]], ssem[nb]).wait()
            else:
                pltpu.make_async_copy(
                    rows[nb], acc_sh.at[cidx.at[nb]], ssem[nb]).wait()

            @pl.when(NBUF * o + b + NBUF - 1 < NCHUNK)
            def _():
                _prep_chunk(nb, nr, nsub)

            pltpu.make_async_copy(
                xwp_hbm.at[ridx.at[b]], rows[b], gsem[b]).wait()
            pltpu.make_async_copy(
                ewp_hbm.at[0, pl.ds(0, CK)], ewr.at[b], esem[b]).wait()

            def grp(i, _):
                for gg in range(2):
                    base = i * 32 + gg * 16
                    ev = ewr[b, pl.ds(base, 16)]
                    for j in range(16):
                        w16 = _bcast_lane(ev, j)
                        k = base + j
                        for d in range(D // 16):
                            rows[b][k, pl.ds(d * 16, 16)] = (
                                rows[b][k, pl.ds(d * 16, 16)] * w16)
                return 0
            lax.fori_loop(0, CK // 32, grp, 0)

            pltpu.async_copy(rows[b], acc_sh.at[cidx.at[b]], ssem[b], add=True)
        return 0
    lax.fori_loop(0, NCHUNK // NBUF, outer, 0)

    pltpu.make_async_copy(
        rows[NBUF - 1], acc_sh.at[cidx.at[NBUF - 1]], ssem[NBUF - 1]).wait()
    plsc.subcore_barrier()

    r0 = sid * ROWS_PER_TILE
    pltpu.sync_copy(acc_sh.at[pl.ds(r0, ROWS_PER_TILE)],
                    part_hbm.at[cid, pl.ds(r0, ROWS_PER_TILE)])


BM = 512
GRID = NPAD // BM


def _mm1_body(x_ref, w_ref, deg_ref, xwp_ref, dis_ref):
    dis = lax.rsqrt(deg_ref[...] + 1.0)
    xw = jnp.dot(x_ref[...], w_ref[...], preferred_element_type=jnp.float32)
    xwp_ref[...] = dis * xw
    dis_ref[...] = dis


_mm1 = pl.pallas_call(
    _mm1_body,
    grid=(GRID,),
    in_specs=[
        pl.BlockSpec((BM, D), lambda i: (i, 0)),
        pl.BlockSpec((D, D), lambda i: (0, 0)),
        pl.BlockSpec((BM, 1), lambda i: (i, 0)),
    ],
    out_specs=[
        pl.BlockSpec((BM, D), lambda i: (i, 0)),
        pl.BlockSpec((BM, 1), lambda i: (i, 0)),
    ],
    out_shape=[
        jax.ShapeDtypeStruct((NPAD, D), jnp.float32),
        jax.ShapeDtypeStruct((NPAD, 1), jnp.float32),
    ],
)


def _mm2_body(p0_ref, p1_ref, xwp_ref, dis_ref, b_ref, w_ref, out_ref):
    dis = dis_ref[...]
    s = dis * (p0_ref[...] + p1_ref[...] + xwp_ref[...]) + b_ref[...]
    h = jnp.maximum(s, 0.0)
    out_ref[...] = dis * jnp.dot(h, w_ref[...], preferred_element_type=jnp.float32)


_mm2 = pl.pallas_call(
    _mm2_body,
    grid=(GRID,),
    in_specs=[
        pl.BlockSpec((BM, D), lambda i: (i, 0)),
        pl.BlockSpec((BM, D), lambda i: (i, 0)),
        pl.BlockSpec((BM, D), lambda i: (i, 0)),
        pl.BlockSpec((BM, 1), lambda i: (i, 0)),
        pl.BlockSpec((1, D), lambda i: (0, 0)),
        pl.BlockSpec((D, D), lambda i: (0, 0)),
    ],
    out_specs=pl.BlockSpec((BM, D), lambda i: (i, 0)),
    out_shape=jax.ShapeDtypeStruct((NPAD, D), jnp.float32),
)


def _fin_body(q0_ref, q1_ref, xwp_ref, dis_ref, b_ref, out_ref):
    s = dis_ref[...] * (q0_ref[...] + q1_ref[...] + xwp_ref[...]) + b_ref[...]
    out_ref[...] = jax.nn.sigmoid(s)


_fin = pl.pallas_call(
    _fin_body,
    grid=(GRID,),
    in_specs=[
        pl.BlockSpec((BM, D), lambda i: (i, 0)),
        pl.BlockSpec((BM, D), lambda i: (i, 0)),
        pl.BlockSpec((BM, D), lambda i: (i, 0)),
        pl.BlockSpec((BM, 1), lambda i: (i, 0)),
        pl.BlockSpec((1, D), lambda i: (0, 0)),
    ],
    out_specs=pl.BlockSpec((BM, D), lambda i: (i, 0)),
    out_shape=jax.ShapeDtypeStruct((NPAD, D), jnp.float32),
)


def kernel(x, edge_index, edge_weight, W1, b1, W2, b2):
    row = edge_index[0].astype(jnp.int32)
    col = edge_index[1].astype(jnp.int32)
    ew = edge_weight.astype(jnp.float32)

    npad_e = EPAD - E
    pad_idx = jnp.arange(npad_e, dtype=jnp.int32) % N
    rowf = jnp.concatenate([row, pad_idx])
    colf = jnp.concatenate([col, pad_idx])
    colp = colf.reshape(EPAD // 128, 128)
    rcp = ((rowf << 14) | colf).reshape(EPAD // 128, 128)
    ewp = jnp.concatenate([ew, jnp.zeros((npad_e,), jnp.float32)]
                          ).reshape(EPAD // 128, 128)
    xpad = jnp.concatenate(
        [x.astype(jnp.float32), jnp.zeros((NPAD - N, D), jnp.float32)])

    deg = _deg_kernel(colp, ewp)
    xwp1, dis = _mm1(xpad, W1, deg.reshape(NPAD, 1))
    p = _agg_kernel(xwp1, rcp, ewp)
    xwp2 = _mm2(p[0], p[1], xwp1, dis, b1.reshape(1, D), W2)
    q = _agg_kernel(xwp2, rcp, ewp)
    out = _fin(q[0], q[1], xwp2, dis, b2.reshape(1, D))
    return out[:N]
